# baseline (speedup 1.0000x reference)
import functools

import jax
import jax.numpy as jnp
from jax import lax
from jax.experimental import pallas as pl
from jax.experimental.pallas import tpu as pltpu
from jax.experimental.pallas import tpu_sc as plsc

N = 10000
E = 320000
D = 128
L = 2

NC = 2
NS = 16
NW = NC * NS
EPW = E // NW
CHUNK = 100
NCHUNK = EPW // CHUNK
NHALF = NCHUNK // 2
P = 10
NPART = NCHUNK // P
RPS = 624
TAIL = N - NS * RPS

_MESH = plsc.VectorSubcoreMesh(core_axis_name="c", subcore_axis_name="s")


def _zero_vmem(ref, rows, width):
    zvec = jnp.zeros((16,), jnp.float32)

    def body(r, _):
        for cc in range(width // 16):
            ref[r, pl.ds(cc * 16, 16)] = zvec
        return 0

    lax.fori_loop(0, rows, body, 0, unroll=False)


def _zero_shared_rows(stage_v, shared, s):
    nfull = RPS // CHUNK
    for k in range(nfull):
        pltpu.sync_copy(stage_v, shared.at[pl.ds(s * RPS + k * CHUNK, CHUNK)])
    rem = RPS - nfull * CHUNK
    if rem:
        pltpu.sync_copy(stage_v.at[pl.ds(0, rem)],
                        shared.at[pl.ds(s * RPS + nfull * CHUNK, rem)])

    @pl.when(s == NS - 1)
    def _():
        pltpu.sync_copy(stage_v.at[pl.ds(0, TAIL)],
                        shared.at[pl.ds(NS * RPS, TAIL)])


def _copy_out_rows(shared, out_hbm, c, s):
    pltpu.sync_copy(shared.at[pl.ds(s * RPS, RPS)],
                    out_hbm.at[c, pl.ds(s * RPS, RPS)])

    @pl.when(s == NS - 1)
    def _():
        pltpu.sync_copy(shared.at[pl.ds(NS * RPS, TAIL)],
                        out_hbm.at[c, pl.ds(NS * RPS, TAIL)])


def _ring_pass(x_hbm, edge_hbm, out_hbm, idx_v, bufs, acc_sh,
               sgs, sss, si, c, s, wid):
    rows0 = bufs[0]
    _zero_vmem(rows0, CHUNK, D)
    _zero_shared_rows(rows0, acc_sh, s)
    plsc.subcore_barrier()

    def src_row(g):
        return idx_v.at[(g // P) % 2, 0, g % P]

    def dst_row(j):
        return idx_v.at[(j // P) % 2, 1, j % P]

    pltpu.sync_copy(edge_hbm.at[wid, 0], idx_v.at[0])
    pltpu.async_copy(edge_hbm.at[wid, 1], idx_v.at[1], si)
    pltpu.async_copy(x_hbm.at[src_row(0)], bufs[0], sgs[0])
    pltpu.async_copy(x_hbm.at[src_row(1)], bufs[1], sgs[1])

    def step(j, _):
        pj = j // P

        @pl.when((j % P == 0) & (j > 0) & (pj + 1 < NPART))
        def _():
            pltpu.async_copy(edge_hbm.at[wid, pj + 1],
                             idx_v.at[(pj + 1) % 2], si)

        for b in range(3):
            b2 = (b + 2) % 3
            onb = j % 3 == b

            @pl.when(onb & (j >= 1))
            def _():
                pltpu.make_async_copy(
                    bufs[b2], acc_sh.at[dst_row(j - 1)], sss[b2]).wait()

            @pl.when(onb & (j % P == P - 2) & (j + 2 < NCHUNK))
            def _():
                pltpu.make_async_copy(edge_hbm.at[wid, (j + 2) // P],
                                      idx_v.at[((j + 2) // P) % 2],
                                      si).wait()

            @pl.when(onb & (j + 2 < NCHUNK))
            def _():
                pltpu.async_copy(x_hbm.at[src_row(j + 2)], bufs[b2], sgs[b2])

            @pl.when(onb)
            def _():
                pltpu.make_async_copy(x_hbm.at[src_row(j)], bufs[b],
                                      sgs[b]).wait()
                pltpu.async_copy(bufs[b], acc_sh.at[dst_row(j)], sss[b],
                                 add=True)

        return 0

    lax.fori_loop(0, NCHUNK, step, 0, unroll=False)

    bl = (NCHUNK - 1) % 3
    pltpu.make_async_copy(bufs[bl], acc_sh.at[dst_row(NCHUNK - 1)],
                          sss[bl]).wait()

    plsc.subcore_barrier()
    _copy_out_rows(acc_sh, out_hbm, c, s)


@functools.partial(
    pl.kernel,
    out_type=jax.ShapeDtypeStruct((NC, N, D), jnp.float32),
    mesh=_MESH,
    scratch_types=[
        pltpu.VMEM((2, 2, P, CHUNK), jnp.int32),
        pltpu.VMEM((CHUNK, D), jnp.float32),
        pltpu.VMEM((CHUNK, D), jnp.float32),
        pltpu.VMEM((CHUNK, D), jnp.float32),
        pltpu.VMEM_SHARED((N, D), jnp.float32),
        pltpu.SemaphoreType.DMA,
        pltpu.SemaphoreType.DMA,
        pltpu.SemaphoreType.DMA,
        pltpu.SemaphoreType.DMA,
        pltpu.SemaphoreType.DMA,
        pltpu.SemaphoreType.DMA,
        pltpu.SemaphoreType.DMA,
    ],
)
def _segment_sum_sc(x_hbm, edge_hbm, out_hbm,
                    idx_v, rows0, rows1, rows2, acc_sh,
                    sg0, sg1, sg2, ss0, ss1, ss2, si):
    c = lax.axis_index("c")
    s = lax.axis_index("s")
    wid = s * NC + c
    _ring_pass(x_hbm, edge_hbm, out_hbm, idx_v, (rows0, rows1, rows2),
               acc_sh, (sg0, sg1, sg2), (ss0, ss1, ss2), si, c, s, wid)


@functools.partial(
    pl.kernel,
    out_type=[jax.ShapeDtypeStruct((NC, N, D), jnp.float32),
              jax.ShapeDtypeStruct((NC, N, D), jnp.float32)],
    mesh=_MESH,
    scratch_types=[
        pltpu.VMEM((2, 2, P, CHUNK), jnp.int32),
        pltpu.VMEM((CHUNK, D), jnp.float32),
        pltpu.VMEM((CHUNK, D), jnp.float32),
        pltpu.VMEM((CHUNK, D), jnp.float32),
        pltpu.VMEM_SHARED((N, D), jnp.float32),
        pltpu.SemaphoreType.DMA,
        pltpu.SemaphoreType.DMA,
        pltpu.SemaphoreType.DMA,
        pltpu.SemaphoreType.DMA,
        pltpu.SemaphoreType.DMA,
        pltpu.SemaphoreType.DMA,
        pltpu.SemaphoreType.DMA,
    ],
)
def _segment_sum2_sc(xa_hbm, xb_hbm, edge_hbm, outa_hbm, outb_hbm,
                     idx_v, rows0, rows1, rows2, acc_sh,
                     sg0, sg1, sg2, ss0, ss1, ss2, si):
    c = lax.axis_index("c")
    s = lax.axis_index("s")
    wid = s * NC + c
    _ring_pass(xa_hbm, edge_hbm, outa_hbm, idx_v, (rows0, rows1, rows2),
               acc_sh, (sg0, sg1, sg2), (ss0, ss1, ss2), si, c, s, wid)
    _ring_pass(xb_hbm, edge_hbm, outb_hbm, idx_v, (rows0, rows1, rows2),
               acc_sh, (sg0, sg1, sg2), (ss0, ss1, ss2), si, c, s, wid)


@functools.partial(
    pl.kernel,
    out_type=jax.ShapeDtypeStruct((NC, N, D), jnp.float32),
    mesh=_MESH,
    scratch_types=[
        pltpu.VMEM((NHALF, CHUNK), jnp.int32),
        pltpu.VMEM((CHUNK, D), jnp.float32),
        pltpu.VMEM_SHARED((N, D), jnp.float32),
        pltpu.SemaphoreType.DMA,
    ],
)
def _degree_sc(dst_hbm, out_hbm, idx_v, ones_v, deg_sh, sem):
    c = lax.axis_index("c")
    s = lax.axis_index("s")
    wid = s * NC + c

    _zero_vmem(ones_v, CHUNK, D)
    _zero_shared_rows(ones_v, deg_sh, s)

    lane = lax.iota(jnp.int32, 16)
    onerow = jnp.where(lane == 0, 1.0, 0.0).astype(jnp.float32)

    def fill(r, _):
        ones_v[r, pl.ds(0, 16)] = onerow
        return 0

    lax.fori_loop(0, CHUNK, fill, 0, unroll=False)
    plsc.subcore_barrier()

    for half in range(2):
        pltpu.sync_copy(dst_hbm.at[wid * 2 + half], idx_v)

        def chunk(j, _):
            pltpu.async_copy(ones_v, deg_sh.at[idx_v.at[j]], sem, add=True)
            return 0

        lax.fori_loop(0, NHALF, chunk, 0, unroll=False)

        def drain(j, _):
            pltpu.make_async_copy(ones_v, deg_sh.at[idx_v.at[0]], sem).wait()
            return 0

        lax.fori_loop(0, NHALF, drain, 0, unroll=False)

    plsc.subcore_barrier()
    _copy_out_rows(deg_sh, out_hbm, c, s)


BN = 1000


def _tc_gates_body(xin_ref, h_ref, aggx_ref, aggh_ref, deg_ref,
                   w_ref, b_ref, z_ref, rh_ref):
    deg = deg_ref[0, :, 0] + deg_ref[1, :, 0]
    inv = 1.0 / jnp.clip(deg, 1.0, None)
    mean_x = (aggx_ref[0] + aggx_ref[1]) * inv[:, None]
    mean_h = (aggh_ref[0] + aggh_ref[1]) * inv[:, None]
    xcat = jnp.concatenate([mean_x, xin_ref[...], mean_h, h_ref[...]], axis=1)
    zr = jnp.dot(xcat, w_ref[...], preferred_element_type=jnp.float32)
    zr = zr + b_ref[...][None, :]
    z = jax.nn.sigmoid(zr[:, :D])
    r = jax.nn.sigmoid(zr[:, D:])
    z_ref[...] = z
    rh_ref[...] = r * h_ref[...]


def _tc_gates(xin, h, aggx, aggh, deg, w, b):
    grid = N // BN
    return pl.pallas_call(
        _tc_gates_body,
        grid=(grid,),
        in_specs=[
            pl.BlockSpec((BN, D), lambda i: (i, 0)),
            pl.BlockSpec((BN, D), lambda i: (i, 0)),
            pl.BlockSpec((NC, BN, D), lambda i: (0, i, 0)),
            pl.BlockSpec((NC, BN, D), lambda i: (0, i, 0)),
            pl.BlockSpec((NC, BN, D), lambda i: (0, i, 0)),
            pl.BlockSpec((4 * D, 2 * D), lambda i: (0, 0)),
            pl.BlockSpec((2 * D,), lambda i: (0,)),
        ],
        out_specs=[
            pl.BlockSpec((BN, D), lambda i: (i, 0)),
            pl.BlockSpec((BN, D), lambda i: (i, 0)),
        ],
        out_shape=[
            jax.ShapeDtypeStruct((N, D), jnp.float32),
            jax.ShapeDtypeStruct((N, D), jnp.float32),
        ],
    )(xin, h, aggx, aggh, deg, w, b)


def _tc_hnew_body(xin_ref, h_ref, rh_ref, z_ref, aggx_ref, aggrh_ref,
                  deg_ref, w_ref, b_ref, out_ref):
    deg = deg_ref[0, :, 0] + deg_ref[1, :, 0]
    inv = 1.0 / jnp.clip(deg, 1.0, None)
    mean_x = (aggx_ref[0] + aggx_ref[1]) * inv[:, None]
    mean_rh = (aggrh_ref[0] + aggrh_ref[1]) * inv[:, None]
    xcat = jnp.concatenate([mean_x, xin_ref[...], mean_rh, rh_ref[...]],
                           axis=1)
    pre = jnp.dot(xcat, w_ref[...], preferred_element_type=jnp.float32)
    h_tilde = jnp.tanh(pre + b_ref[...][None, :])
    z = z_ref[...]
    out_ref[...] = z * h_ref[...] + (1.0 - z) * h_tilde


def _tc_hnew(xin, h, rh, z, aggx, aggrh, deg, w, b):
    grid = N // BN
    return pl.pallas_call(
        _tc_hnew_body,
        grid=(grid,),
        in_specs=[
            pl.BlockSpec((BN, D), lambda i: (i, 0)),
            pl.BlockSpec((BN, D), lambda i: (i, 0)),
            pl.BlockSpec((BN, D), lambda i: (i, 0)),
            pl.BlockSpec((BN, D), lambda i: (i, 0)),
            pl.BlockSpec((NC, BN, D), lambda i: (0, i, 0)),
            pl.BlockSpec((NC, BN, D), lambda i: (0, i, 0)),
            pl.BlockSpec((NC, BN, D), lambda i: (0, i, 0)),
            pl.BlockSpec((4 * D, D), lambda i: (0, 0)),
            pl.BlockSpec((D,), lambda i: (0,)),
        ],
        out_specs=pl.BlockSpec((BN, D), lambda i: (i, 0)),
        out_shape=jax.ShapeDtypeStruct((N, D), jnp.float32),
    )(xin, h, rh, z, aggx, aggrh, deg, w, b)


def kernel(inp, edgidx, h, Wl, Wr, b):
    edges = edgidx.reshape(2, NW, NPART, P, CHUNK).transpose(1, 2, 0, 3, 4)
    dst = edgidx[1].reshape(NW * 2, NHALF, CHUNK)

    deg2 = _degree_sc(dst)

    h_out = []
    xin = inp
    for i in range(L):
        hi = h[i]
        wz = jnp.concatenate([Wl[i, 0], Wr[i, 0], Wl[i, 1], Wr[i, 1]], axis=0)
        wr_ = jnp.concatenate([Wl[i, 2], Wr[i, 2], Wl[i, 3], Wr[i, 3]], axis=0)
        wzr = jnp.concatenate([wz, wr_], axis=1)
        bzr = jnp.concatenate([b[i, 0] + b[i, 1], b[i, 2] + b[i, 3]], axis=0)
        wh = jnp.concatenate([Wl[i, 4], Wr[i, 4], Wl[i, 5], Wr[i, 5]], axis=0)
        bh = b[i, 4] + b[i, 5]

        aggx, aggh = _segment_sum2_sc(xin, hi, edges)
        z, rh = _tc_gates(xin, hi, aggx, aggh, deg2, wzr, bzr)
        aggrh = _segment_sum_sc(rh, edges)
        hn = _tc_hnew(xin, hi, rh, z, aggx, aggrh, deg2, wh, bh)
        h_out.append(hn)
        xin = hn

    out = jnp.stack(h_out, axis=0)
    return (out, out)


# speedup vs baseline: 1.0060x; 1.0060x over previous
import functools

import jax
import jax.numpy as jnp
from jax import lax
from jax.experimental import pallas as pl
from jax.experimental.pallas import tpu as pltpu
from jax.experimental.pallas import tpu_sc as plsc

N = 10000
E = 320000
D = 128
L = 2

NC = 2
NS = 16
NW = NC * NS
EPW = E // NW
CHUNK = 100
NCHUNK = EPW // CHUNK
NHALF = NCHUNK // 2
P = 10
NPART = NCHUNK // P
RPS = 624
TAIL = N - NS * RPS

_MESH = plsc.VectorSubcoreMesh(core_axis_name="c", subcore_axis_name="s")


def _zero_vmem(ref, rows, width):
    zvec = jnp.zeros((16,), jnp.float32)

    def body(r, _):
        for cc in range(width // 16):
            ref[r, pl.ds(cc * 16, 16)] = zvec
        return 0

    lax.fori_loop(0, rows, body, 0, unroll=False)


def _zero_shared_rows(stage_v, shared, s):
    nfull = RPS // CHUNK
    for k in range(nfull):
        pltpu.sync_copy(stage_v, shared.at[pl.ds(s * RPS + k * CHUNK, CHUNK)])
    rem = RPS - nfull * CHUNK
    if rem:
        pltpu.sync_copy(stage_v.at[pl.ds(0, rem)],
                        shared.at[pl.ds(s * RPS + nfull * CHUNK, rem)])

    @pl.when(s == NS - 1)
    def _():
        pltpu.sync_copy(stage_v.at[pl.ds(0, TAIL)],
                        shared.at[pl.ds(NS * RPS, TAIL)])


def _copy_out_rows(shared, out_hbm, c, s):
    pltpu.sync_copy(shared.at[pl.ds(s * RPS, RPS)],
                    out_hbm.at[c, pl.ds(s * RPS, RPS)])

    @pl.when(s == NS - 1)
    def _():
        pltpu.sync_copy(shared.at[pl.ds(NS * RPS, TAIL)],
                        out_hbm.at[c, pl.ds(NS * RPS, TAIL)])


def _ring_pass(x_hbm, edge_hbm, out_hbm, idx_v, bufs, acc_sh,
               sgs, sss, si, c, s, wid):
    rows0 = bufs[0]
    _zero_vmem(rows0, CHUNK, D)
    _zero_shared_rows(rows0, acc_sh, s)
    plsc.subcore_barrier()

    def src_row(g):
        return idx_v.at[(g // P) % 2, 0, g % P]

    def dst_row(j):
        return idx_v.at[(j // P) % 2, 1, j % P]

    pltpu.sync_copy(edge_hbm.at[wid, 0], idx_v.at[0])
    pltpu.async_copy(edge_hbm.at[wid, 1], idx_v.at[1], si)
    pltpu.async_copy(x_hbm.at[src_row(0)], bufs[0], sgs[0])
    pltpu.async_copy(x_hbm.at[src_row(1)], bufs[1], sgs[1])

    def step(j, _):
        pj = j // P

        @pl.when((j % P == 0) & (j > 0) & (pj + 1 < NPART))
        def _():
            pltpu.async_copy(edge_hbm.at[wid, pj + 1],
                             idx_v.at[(pj + 1) % 2], si)

        for b in range(3):
            b2 = (b + 2) % 3
            onb = j % 3 == b

            @pl.when(onb & (j >= 1))
            def _():
                pltpu.make_async_copy(
                    bufs[b2], acc_sh.at[dst_row(j - 1)], sss[b2]).wait()

            @pl.when(onb & (j % P == P - 2) & (j + 2 < NCHUNK))
            def _():
                pltpu.make_async_copy(edge_hbm.at[wid, (j + 2) // P],
                                      idx_v.at[((j + 2) // P) % 2],
                                      si).wait()

            @pl.when(onb & (j + 2 < NCHUNK))
            def _():
                pltpu.async_copy(x_hbm.at[src_row(j + 2)], bufs[b2], sgs[b2])

            @pl.when(onb)
            def _():
                pltpu.make_async_copy(x_hbm.at[src_row(j)], bufs[b],
                                      sgs[b]).wait()
                pltpu.async_copy(bufs[b], acc_sh.at[dst_row(j)], sss[b],
                                 add=True)

        return 0

    lax.fori_loop(0, NCHUNK, step, 0, unroll=False)

    bl = (NCHUNK - 1) % 3
    pltpu.make_async_copy(bufs[bl], acc_sh.at[dst_row(NCHUNK - 1)],
                          sss[bl]).wait()

    plsc.subcore_barrier()
    _copy_out_rows(acc_sh, out_hbm, c, s)


@functools.partial(
    pl.kernel,
    out_type=jax.ShapeDtypeStruct((NC, N, D), jnp.float32),
    mesh=_MESH,
    scratch_types=[
        pltpu.VMEM((2, 2, P, CHUNK), jnp.int32),
        pltpu.VMEM((CHUNK, D), jnp.float32),
        pltpu.VMEM((CHUNK, D), jnp.float32),
        pltpu.VMEM((CHUNK, D), jnp.float32),
        pltpu.VMEM_SHARED((N, D), jnp.float32),
        pltpu.SemaphoreType.DMA,
        pltpu.SemaphoreType.DMA,
        pltpu.SemaphoreType.DMA,
        pltpu.SemaphoreType.DMA,
        pltpu.SemaphoreType.DMA,
        pltpu.SemaphoreType.DMA,
        pltpu.SemaphoreType.DMA,
    ],
)
def _segment_sum_sc(x_hbm, edge_hbm, out_hbm,
                    idx_v, rows0, rows1, rows2, acc_sh,
                    sg0, sg1, sg2, ss0, ss1, ss2, si):
    c = lax.axis_index("c")
    s = lax.axis_index("s")
    wid = s * NC + c
    _ring_pass(x_hbm, edge_hbm, out_hbm, idx_v, (rows0, rows1, rows2),
               acc_sh, (sg0, sg1, sg2), (ss0, ss1, ss2), si, c, s, wid)


@functools.partial(
    pl.kernel,
    out_type=jax.ShapeDtypeStruct((NC, N, D), jnp.float32),
    mesh=_MESH,
    scratch_types=[
        pltpu.VMEM((NHALF, CHUNK), jnp.int32),
        pltpu.VMEM((CHUNK, D), jnp.float32),
        pltpu.VMEM_SHARED((N, D), jnp.float32),
        pltpu.SemaphoreType.DMA,
    ],
)
def _degree_sc(dst_hbm, out_hbm, idx_v, ones_v, deg_sh, sem):
    c = lax.axis_index("c")
    s = lax.axis_index("s")
    wid = s * NC + c

    _zero_vmem(ones_v, CHUNK, D)
    _zero_shared_rows(ones_v, deg_sh, s)

    lane = lax.iota(jnp.int32, 16)
    onerow = jnp.where(lane == 0, 1.0, 0.0).astype(jnp.float32)

    def fill(r, _):
        ones_v[r, pl.ds(0, 16)] = onerow
        return 0

    lax.fori_loop(0, CHUNK, fill, 0, unroll=False)
    plsc.subcore_barrier()

    for half in range(2):
        pltpu.sync_copy(dst_hbm.at[wid * 2 + half], idx_v)

        def chunk(j, _):
            pltpu.async_copy(ones_v, deg_sh.at[idx_v.at[j]], sem, add=True)
            return 0

        lax.fori_loop(0, NHALF, chunk, 0, unroll=False)

        def drain(j, _):
            pltpu.make_async_copy(ones_v, deg_sh.at[idx_v.at[0]], sem).wait()
            return 0

        lax.fori_loop(0, NHALF, drain, 0, unroll=False)

    plsc.subcore_barrier()
    _copy_out_rows(deg_sh, out_hbm, c, s)


BN = 1000


def _tc_gates_body(xin_ref, h_ref, aggx_ref, aggh_ref, deg_ref,
                   w_ref, b_ref, z_ref, rh_ref):
    deg = deg_ref[0, :, 0] + deg_ref[1, :, 0]
    inv = 1.0 / jnp.clip(deg, 1.0, None)
    mean_x = (aggx_ref[0] + aggx_ref[1]) * inv[:, None]
    mean_h = (aggh_ref[0] + aggh_ref[1]) * inv[:, None]
    xcat = jnp.concatenate([mean_x, xin_ref[...], mean_h, h_ref[...]], axis=1)
    zr = jnp.dot(xcat, w_ref[...], preferred_element_type=jnp.float32)
    zr = zr + b_ref[...][None, :]
    z = jax.nn.sigmoid(zr[:, :D])
    r = jax.nn.sigmoid(zr[:, D:])
    z_ref[...] = z
    rh_ref[...] = r * h_ref[...]


def _tc_gates(xin, h, aggx, aggh, deg, w, b):
    grid = N // BN
    return pl.pallas_call(
        _tc_gates_body,
        grid=(grid,),
        in_specs=[
            pl.BlockSpec((BN, D), lambda i: (i, 0)),
            pl.BlockSpec((BN, D), lambda i: (i, 0)),
            pl.BlockSpec((NC, BN, D), lambda i: (0, i, 0)),
            pl.BlockSpec((NC, BN, D), lambda i: (0, i, 0)),
            pl.BlockSpec((NC, BN, D), lambda i: (0, i, 0)),
            pl.BlockSpec((4 * D, 2 * D), lambda i: (0, 0)),
            pl.BlockSpec((2 * D,), lambda i: (0,)),
        ],
        out_specs=[
            pl.BlockSpec((BN, D), lambda i: (i, 0)),
            pl.BlockSpec((BN, D), lambda i: (i, 0)),
        ],
        out_shape=[
            jax.ShapeDtypeStruct((N, D), jnp.float32),
            jax.ShapeDtypeStruct((N, D), jnp.float32),
        ],
    )(xin, h, aggx, aggh, deg, w, b)


def _tc_hnew_body(xin_ref, h_ref, rh_ref, z_ref, aggx_ref, aggrh_ref,
                  deg_ref, w_ref, b_ref, out_ref):
    deg = deg_ref[0, :, 0] + deg_ref[1, :, 0]
    inv = 1.0 / jnp.clip(deg, 1.0, None)
    mean_x = (aggx_ref[0] + aggx_ref[1]) * inv[:, None]
    mean_rh = (aggrh_ref[0] + aggrh_ref[1]) * inv[:, None]
    xcat = jnp.concatenate([mean_x, xin_ref[...], mean_rh, rh_ref[...]],
                           axis=1)
    pre = jnp.dot(xcat, w_ref[...], preferred_element_type=jnp.float32)
    h_tilde = jnp.tanh(pre + b_ref[...][None, :])
    z = z_ref[...]
    out_ref[...] = z * h_ref[...] + (1.0 - z) * h_tilde


def _tc_hnew(xin, h, rh, z, aggx, aggrh, deg, w, b):
    grid = N // BN
    return pl.pallas_call(
        _tc_hnew_body,
        grid=(grid,),
        in_specs=[
            pl.BlockSpec((BN, D), lambda i: (i, 0)),
            pl.BlockSpec((BN, D), lambda i: (i, 0)),
            pl.BlockSpec((BN, D), lambda i: (i, 0)),
            pl.BlockSpec((BN, D), lambda i: (i, 0)),
            pl.BlockSpec((NC, BN, D), lambda i: (0, i, 0)),
            pl.BlockSpec((NC, BN, D), lambda i: (0, i, 0)),
            pl.BlockSpec((NC, BN, D), lambda i: (0, i, 0)),
            pl.BlockSpec((4 * D, D), lambda i: (0, 0)),
            pl.BlockSpec((D,), lambda i: (0,)),
        ],
        out_specs=pl.BlockSpec((BN, D), lambda i: (i, 0)),
        out_shape=jax.ShapeDtypeStruct((N, D), jnp.float32),
    )(xin, h, rh, z, aggx, aggrh, deg, w, b)


def kernel(inp, edgidx, h, Wl, Wr, b):
    edges = edgidx.reshape(2, NW, NPART, P, CHUNK).transpose(1, 2, 0, 3, 4)
    dst = edgidx[1].reshape(NW * 2, NHALF, CHUNK)

    deg2 = _degree_sc(dst)

    h_out = []
    xin = inp
    for i in range(L):
        hi = h[i]
        wz = jnp.concatenate([Wl[i, 0], Wr[i, 0], Wl[i, 1], Wr[i, 1]], axis=0)
        wr_ = jnp.concatenate([Wl[i, 2], Wr[i, 2], Wl[i, 3], Wr[i, 3]], axis=0)
        wzr = jnp.concatenate([wz, wr_], axis=1)
        bzr = jnp.concatenate([b[i, 0] + b[i, 1], b[i, 2] + b[i, 3]], axis=0)
        wh = jnp.concatenate([Wl[i, 4], Wr[i, 4], Wl[i, 5], Wr[i, 5]], axis=0)
        bh = b[i, 4] + b[i, 5]

        aggx = _segment_sum_sc(xin, edges)
        aggh = _segment_sum_sc(hi, edges)
        z, rh = _tc_gates(xin, hi, aggx, aggh, deg2, wzr, bzr)
        aggrh = _segment_sum_sc(rh, edges)
        hn = _tc_hnew(xin, hi, rh, z, aggx, aggrh, deg2, wh, bh)
        h_out.append(hn)
        xin = hn

    out = jnp.stack(h_out, axis=0)
    return (out, out)
